# baseline (device time: 91894 ns/iter reference)
import jax
import jax.numpy as jnp
from jax import lax
from jax.experimental import pallas as pl
from jax.experimental.pallas import tpu as pltpu

B, H, D, BS = 8, 8, 128, 16
NB_LOCAL = 512
NTOK = NB_LOCAL * BS
CHUNK = 64
NCHUNK = NB_LOCAL // CHUNK
CTOK = CHUNK * BS
SCALE = D ** -0.5
NEG = -1e30


def kernel(Q, K, V, bt, lens):
    my_x = lax.axis_index("x")

    page_ids = my_x * NB_LOCAL + jnp.arange(NB_LOCAL, dtype=jnp.int32)
    valid = jnp.arange(bt.shape[1], dtype=jnp.int32)[None, :] < lens[:, None]
    cnt = jnp.sum(
        (bt[:, :, None] == page_ids[None, None, :]) & valid[:, :, None],
        axis=1,
        dtype=jnp.int32,
    )
    w = jnp.repeat(cnt.astype(jnp.float32), BS, axis=1)

    def body(q_ref, k_ref, v_ref, w_ref, out_ref,
             m_acc, l_acc, o_acc, recv_m, recv_l, recv_o,
             send_sems, recv_sems):
        c = pl.program_id(0)

        @pl.when(c == 0)
        def _():
            m_acc[...] = jnp.full((B, H), NEG, jnp.float32)
            l_acc[...] = jnp.zeros((B, H), jnp.float32)
            o_acc[...] = jnp.zeros((B, H, D), jnp.float32)

        wv = w_ref[...]
        for h in range(H):
            qh = q_ref[:, 0, h, :].astype(jnp.bfloat16)
            kh = k_ref[:, :, h, :].reshape(CTOK, D).astype(jnp.bfloat16)
            vh = v_ref[:, :, h, :].reshape(CTOK, D).astype(jnp.bfloat16)

            s = lax.dot_general(
                qh, kh, (((1,), (1,)), ((), ())),
                preferred_element_type=jnp.float32,
            ) * SCALE
            s = jnp.where(wv > 0.0, s, NEG)

            m_old = m_acc[:, h:h + 1]
            m_new = jnp.maximum(m_old, jnp.max(s, axis=1, keepdims=True))
            alpha = jnp.exp(m_old - m_new)
            p = wv * jnp.exp(s - m_new)
            l_new = alpha * l_acc[:, h:h + 1] + jnp.sum(p, axis=1, keepdims=True)
            pv = lax.dot_general(
                p.astype(jnp.bfloat16), vh, (((1,), (0,)), ((), ())),
                preferred_element_type=jnp.float32,
            )
            o_acc[:, h, :] = alpha * o_acc[:, h, :] + pv
            m_acc[:, h:h + 1] = m_new
            l_acc[:, h:h + 1] = l_new

        @pl.when(c == NCHUNK - 1)
        def _():
            x = lax.axis_index("x")
            y = lax.axis_index("y")
            z = lax.axis_index("z")
            partner = (1 - x, y, z)

            bsem = pltpu.get_barrier_semaphore()
            pl.semaphore_signal(
                bsem, inc=1,
                device_id=partner, device_id_type=pl.DeviceIdType.MESH,
            )
            pl.semaphore_wait(bsem, 1)

            rdmas = [
                pltpu.make_async_remote_copy(
                    src_ref=src, dst_ref=dst,
                    send_sem=send_sems.at[i], recv_sem=recv_sems.at[i],
                    device_id=partner, device_id_type=pl.DeviceIdType.MESH,
                )
                for i, (src, dst) in enumerate(
                    [(m_acc, recv_m), (l_acc, recv_l), (o_acc, recv_o)]
                )
            ]
            for r in rdmas:
                r.start()
            for r in rdmas:
                r.wait()

            m_l, l_l, o_l = m_acc[...], l_acc[...], o_acc[...]
            m_r, l_r, o_r = recv_m[...], recv_l[...], recv_o[...]

            mx = jnp.maximum(m_l, m_r)
            a = jnp.exp(m_l - mx)
            b = jnp.exp(m_r - mx)
            lsum = a * l_l + b * l_r
            out = (a[:, :, None] * o_l + b[:, :, None] * o_r) / lsum[:, :, None]
            out_ref[:, 0, :, :] = out

    return pl.pallas_call(
        body,
        grid=(NCHUNK,),
        out_shape=jax.ShapeDtypeStruct((B, 1, H, D), jnp.float32),
        in_specs=[
            pl.BlockSpec((B, 1, H, D), lambda c: (0, 0, 0, 0)),
            pl.BlockSpec((CHUNK, BS, H, D), lambda c: (c, 0, 0, 0)),
            pl.BlockSpec((CHUNK, BS, H, D), lambda c: (c, 0, 0, 0)),
            pl.BlockSpec((B, CTOK), lambda c: (0, c)),
        ],
        out_specs=pl.BlockSpec((B, 1, H, D), lambda c: (0, 0, 0, 0)),
        scratch_shapes=[
            pltpu.VMEM((B, H), jnp.float32),
            pltpu.VMEM((B, H), jnp.float32),
            pltpu.VMEM((B, H, D), jnp.float32),
            pltpu.VMEM((B, H), jnp.float32),
            pltpu.VMEM((B, H), jnp.float32),
            pltpu.VMEM((B, H, D), jnp.float32),
            pltpu.SemaphoreType.DMA((3,)),
            pltpu.SemaphoreType.DMA((3,)),
        ],
        compiler_params=pltpu.CompilerParams(
            dimension_semantics=("arbitrary",),
            collective_id=0,
        ),
    )(Q, K, V, w)


# device time: 37350 ns/iter; 2.4603x vs baseline; 2.4603x over previous
import jax
import jax.numpy as jnp
from jax import lax
from jax.experimental import pallas as pl
from jax.experimental.pallas import tpu as pltpu

B, H, D, BS = 8, 8, 128, 16
NB_LOCAL = 512
NTOK = NB_LOCAL * BS
CHUNK = 64
NCHUNK = NB_LOCAL // CHUNK
CTOK = CHUNK * BS
HL = 2
SCALE = D ** -0.5
NEG = -1e30


def kernel(Q, K, V, bt, lens):
    my_x = lax.axis_index("x")

    page_ids = my_x * NB_LOCAL + jnp.arange(NB_LOCAL, dtype=jnp.int32)
    valid = jnp.arange(bt.shape[1], dtype=jnp.int32)[None, :] < lens[:, None]
    cnt = jnp.sum(
        (bt[:, :, None] == page_ids[None, None, :]) & valid[:, :, None],
        axis=1,
        dtype=jnp.int32,
    )
    w = jnp.repeat(cnt.astype(jnp.float32), BS, axis=1)

    def body(q_ref, k_ref, v_ref, w_ref, out_ref,
             m_acc, l_acc, o_acc, recv_m, recv_l, recv_o, gather_buf,
             xsend_sems, xrecv_sems, gsend_sems, grecv_sems):
        c = pl.program_id(0)
        y = lax.axis_index("y")
        z = lax.axis_index("z")

        @pl.when(c == 0)
        def _():
            m_acc[...] = jnp.full((B, HL), NEG, jnp.float32)
            l_acc[...] = jnp.zeros((B, HL), jnp.float32)
            o_acc[...] = jnp.zeros((B, HL, D), jnp.float32)

        wv = w_ref[...]

        for yy in range(2):
            for zz in range(2):
                @pl.when((y == yy) & (z == zz))
                def _(yy=yy, zz=zz):
                    h0 = (yy * 2 + zz) * HL
                    for hi in range(HL):
                        h = h0 + hi
                        qh = q_ref[:, 0, h, :]
                        kh = k_ref[:, :, h, :].reshape(CTOK, D)
                        vh = v_ref[:, :, h, :].reshape(CTOK, D)

                        s = lax.dot_general(
                            qh, kh, (((1,), (1,)), ((), ())),
                            preferred_element_type=jnp.float32,
                        ) * SCALE
                        s = jnp.where(wv > 0.0, s, NEG)

                        m_old = m_acc[:, hi:hi + 1]
                        m_new = jnp.maximum(
                            m_old, jnp.max(s, axis=1, keepdims=True))
                        alpha = jnp.exp(m_old - m_new)
                        p = wv * jnp.exp(s - m_new)
                        l_new = (alpha * l_acc[:, hi:hi + 1]
                                 + jnp.sum(p, axis=1, keepdims=True))
                        pv = lax.dot_general(
                            p, vh, (((1,), (0,)), ((), ())),
                            preferred_element_type=jnp.float32,
                        )
                        o_acc[:, hi, :] = alpha * o_acc[:, hi, :] + pv
                        m_acc[:, hi:hi + 1] = m_new
                        l_acc[:, hi:hi + 1] = l_new

        @pl.when(c == NCHUNK - 1)
        def _():
            x = lax.axis_index("x")

            bsem = pltpu.get_barrier_semaphore()
            partners = [
                (1 - x, y, z),
                (x, 1 - y, z),
                (x, y, 1 - z),
                (x, 1 - y, 1 - z),
            ]
            for dev in partners:
                pl.semaphore_signal(
                    bsem, inc=1,
                    device_id=dev, device_id_type=pl.DeviceIdType.MESH,
                )
            pl.semaphore_wait(bsem, len(partners))

            xrdmas = [
                pltpu.make_async_remote_copy(
                    src_ref=src, dst_ref=dst,
                    send_sem=xsend_sems.at[i], recv_sem=xrecv_sems.at[i],
                    device_id=(1 - x, y, z),
                    device_id_type=pl.DeviceIdType.MESH,
                )
                for i, (src, dst) in enumerate(
                    [(m_acc, recv_m), (l_acc, recv_l), (o_acc, recv_o)]
                )
            ]
            for r in xrdmas:
                r.start()
            for r in xrdmas:
                r.wait()

            m_l, l_l, o_l = m_acc[...], l_acc[...], o_acc[...]
            m_r, l_r, o_r = recv_m[...], recv_l[...], recv_o[...]

            mx = jnp.maximum(m_l, m_r)
            a = jnp.exp(m_l - mx)
            b = jnp.exp(m_r - mx)
            lsum = a * l_l + b * l_r
            quarter = (a[:, :, None] * o_l + b[:, :, None] * o_r) \
                / lsum[:, :, None]

            for yy in range(2):
                for zz in range(2):
                    @pl.when((y == yy) & (z == zz))
                    def _(yy=yy, zz=zz):
                        my_q = yy * 2 + zz
                        gather_buf[my_q, :, :, :] = quarter
                        grdmas = []
                        others = [
                            (y2, z2)
                            for y2 in range(2) for z2 in range(2)
                            if (y2, z2) != (yy, zz)
                        ]
                        for i, (y2, z2) in enumerate(others):
                            grdmas.append(pltpu.make_async_remote_copy(
                                src_ref=gather_buf.at[my_q],
                                dst_ref=gather_buf.at[my_q],
                                send_sem=gsend_sems.at[i],
                                recv_sem=grecv_sems.at[my_q],
                                device_id=(x, y2, z2),
                                device_id_type=pl.DeviceIdType.MESH,
                            ))
                        for r in grdmas:
                            r.start()
                        for r in grdmas:
                            r.wait_send()
                        for q2 in range(4):
                            if q2 != my_q:
                                rr = pltpu.make_async_remote_copy(
                                    src_ref=gather_buf.at[my_q],
                                    dst_ref=gather_buf.at[q2],
                                    send_sem=gsend_sems.at[0],
                                    recv_sem=grecv_sems.at[q2],
                                    device_id=(x, yy, zz),
                                    device_id_type=pl.DeviceIdType.MESH,
                                )
                                rr.wait_recv()

            for q in range(4):
                out_ref[:, 0, q * HL:(q + 1) * HL, :] = gather_buf[q]

    return pl.pallas_call(
        body,
        grid=(NCHUNK,),
        out_shape=jax.ShapeDtypeStruct((B, 1, H, D), jnp.float32),
        in_specs=[
            pl.BlockSpec((B, 1, H, D), lambda c: (0, 0, 0, 0)),
            pl.BlockSpec((CHUNK, BS, H, D), lambda c: (c, 0, 0, 0)),
            pl.BlockSpec((CHUNK, BS, H, D), lambda c: (c, 0, 0, 0)),
            pl.BlockSpec((B, CTOK), lambda c: (0, c)),
        ],
        out_specs=pl.BlockSpec((B, 1, H, D), lambda c: (0, 0, 0, 0)),
        scratch_shapes=[
            pltpu.VMEM((B, HL), jnp.float32),
            pltpu.VMEM((B, HL), jnp.float32),
            pltpu.VMEM((B, HL, D), jnp.float32),
            pltpu.VMEM((B, HL), jnp.float32),
            pltpu.VMEM((B, HL), jnp.float32),
            pltpu.VMEM((B, HL, D), jnp.float32),
            pltpu.VMEM((4, B, HL, D), jnp.float32),
            pltpu.SemaphoreType.DMA((3,)),
            pltpu.SemaphoreType.DMA((3,)),
            pltpu.SemaphoreType.DMA((3,)),
            pltpu.SemaphoreType.DMA((4,)),
        ],
        compiler_params=pltpu.CompilerParams(
            dimension_semantics=("arbitrary",),
            collective_id=0,
        ),
    )(Q, K, V, w)


# device time: 28674 ns/iter; 3.2048x vs baseline; 1.3026x over previous
import jax
import jax.numpy as jnp
from jax import lax
from jax.experimental import pallas as pl
from jax.experimental.pallas import tpu as pltpu

B, H, D, BS = 8, 8, 128, 16
NB_LOCAL = 512
NTOK = NB_LOCAL * BS
CHUNK = 64
NCHUNK = NB_LOCAL // CHUNK
CTOK = CHUNK * BS
HL = 2
SCALE = D ** -0.5
NEG = -1e30


def kernel(Q, K, V, bt, lens):
    my_x = lax.axis_index("x")

    page_ids = my_x * NB_LOCAL + jnp.arange(NB_LOCAL, dtype=jnp.int32)
    valid = jnp.arange(bt.shape[1], dtype=jnp.int32)[None, :] < lens[:, None]
    cnt = jnp.sum(
        (bt[:, :, None] == page_ids[None, None, :]) & valid[:, :, None],
        axis=1,
        dtype=jnp.int32,
    )
    w = jnp.repeat(cnt.astype(jnp.float32), BS, axis=1)

    def body(q_ref, k_hbm, v_hbm, w_ref, out_ref,
             kbuf, vbuf, qbuf, m_acc, l_acc, o_acc,
             recv_m, recv_l, recv_o, gather_buf,
             ksems, vsems, qsem,
             xsend_sems, xrecv_sems, gsend_sems, grecv_sems):
        y = lax.axis_index("y")
        z = lax.axis_index("z")
        h0 = (y * 2 + z) * HL

        descs = {}

        def start_copy(ci):
            slot = ci % 2
            dk = pltpu.make_async_copy(
                k_hbm.at[pl.ds(ci * CHUNK, CHUNK), :, pl.ds(h0, HL), :],
                kbuf.at[slot], ksems.at[slot])
            dv = pltpu.make_async_copy(
                v_hbm.at[pl.ds(ci * CHUNK, CHUNK), :, pl.ds(h0, HL), :],
                vbuf.at[slot], vsems.at[slot])
            dk.start()
            dv.start()
            descs[ci] = (dk, dv)

        m_acc[...] = jnp.full((B, HL), NEG, jnp.float32)
        l_acc[...] = jnp.zeros((B, HL), jnp.float32)
        o_acc[...] = jnp.zeros((B, HL, D), jnp.float32)

        qcopy = pltpu.make_async_copy(
            q_ref.at[:, :, pl.ds(h0, HL), :], qbuf, qsem)
        qcopy.start()

        start_copy(0)
        qcopy.wait()
        for ci in range(NCHUNK):
            if ci + 1 < NCHUNK:
                start_copy(ci + 1)
            dk, dv = descs[ci]
            dk.wait()
            dv.wait()
            slot = ci % 2

            wv = w_ref[:, ci * CTOK:(ci + 1) * CTOK]
            for hi in range(HL):
                qh = qbuf[:, 0, hi, :]
                kh = kbuf[slot, :, :, hi, :].reshape(CTOK, D)
                vh = vbuf[slot, :, :, hi, :].reshape(CTOK, D)

                s = lax.dot_general(
                    qh, kh, (((1,), (1,)), ((), ())),
                    preferred_element_type=jnp.float32,
                ) * SCALE
                s = jnp.where(wv > 0.0, s, NEG)

                m_old = m_acc[:, hi:hi + 1]
                m_new = jnp.maximum(m_old, jnp.max(s, axis=1, keepdims=True))
                alpha = jnp.exp(m_old - m_new)
                p = wv * jnp.exp(s - m_new)
                l_new = (alpha * l_acc[:, hi:hi + 1]
                         + jnp.sum(p, axis=1, keepdims=True))
                pv = lax.dot_general(
                    p, vh, (((1,), (0,)), ((), ())),
                    preferred_element_type=jnp.float32,
                )
                o_acc[:, hi, :] = alpha * o_acc[:, hi, :] + pv
                m_acc[:, hi:hi + 1] = m_new
                l_acc[:, hi:hi + 1] = l_new

        x = lax.axis_index("x")

        bsem = pltpu.get_barrier_semaphore()
        partners = [
            (1 - x, y, z),
            (x, 1 - y, z),
            (x, y, 1 - z),
            (x, 1 - y, 1 - z),
        ]
        for dev in partners:
            pl.semaphore_signal(
                bsem, inc=1,
                device_id=dev, device_id_type=pl.DeviceIdType.MESH,
            )
        pl.semaphore_wait(bsem, len(partners))

        xrdmas = [
            pltpu.make_async_remote_copy(
                src_ref=src, dst_ref=dst,
                send_sem=xsend_sems.at[i], recv_sem=xrecv_sems.at[i],
                device_id=(1 - x, y, z),
                device_id_type=pl.DeviceIdType.MESH,
            )
            for i, (src, dst) in enumerate(
                [(m_acc, recv_m), (l_acc, recv_l), (o_acc, recv_o)]
            )
        ]
        for r in xrdmas:
            r.start()
        for r in xrdmas:
            r.wait()

        m_l, l_l, o_l = m_acc[...], l_acc[...], o_acc[...]
        m_r, l_r, o_r = recv_m[...], recv_l[...], recv_o[...]

        mx = jnp.maximum(m_l, m_r)
        a = jnp.exp(m_l - mx)
        b = jnp.exp(m_r - mx)
        lsum = a * l_l + b * l_r
        quarter = (a[:, :, None] * o_l + b[:, :, None] * o_r) \
            / lsum[:, :, None]

        for yy in range(2):
            for zz in range(2):
                @pl.when((y == yy) & (z == zz))
                def _(yy=yy, zz=zz):
                    my_q = yy * 2 + zz
                    gather_buf[my_q, :, :, :] = quarter
                    grdmas = []
                    others = [
                        (y2, z2)
                        for y2 in range(2) for z2 in range(2)
                        if (y2, z2) != (yy, zz)
                    ]
                    for i, (y2, z2) in enumerate(others):
                        grdmas.append(pltpu.make_async_remote_copy(
                            src_ref=gather_buf.at[my_q],
                            dst_ref=gather_buf.at[my_q],
                            send_sem=gsend_sems.at[i],
                            recv_sem=grecv_sems.at[my_q],
                            device_id=(x, y2, z2),
                            device_id_type=pl.DeviceIdType.MESH,
                        ))
                    for r in grdmas:
                        r.start()
                    for r in grdmas:
                        r.wait_send()
                    for q2 in range(4):
                        if q2 != my_q:
                            rr = pltpu.make_async_remote_copy(
                                src_ref=gather_buf.at[my_q],
                                dst_ref=gather_buf.at[q2],
                                send_sem=gsend_sems.at[0],
                                recv_sem=grecv_sems.at[q2],
                                device_id=(x, yy, zz),
                                device_id_type=pl.DeviceIdType.MESH,
                            )
                            rr.wait_recv()

        for q in range(4):
            out_ref[:, 0, q * HL:(q + 1) * HL, :] = gather_buf[q]

    return pl.pallas_call(
        body,
        out_shape=jax.ShapeDtypeStruct((B, 1, H, D), jnp.float32),
        in_specs=[
            pl.BlockSpec(memory_space=pltpu.VMEM),
            pl.BlockSpec(memory_space=pl.ANY),
            pl.BlockSpec(memory_space=pl.ANY),
            pl.BlockSpec(memory_space=pltpu.VMEM),
        ],
        out_specs=pl.BlockSpec(memory_space=pltpu.VMEM),
        scratch_shapes=[
            pltpu.VMEM((2, CHUNK, BS, HL, D), jnp.float32),
            pltpu.VMEM((2, CHUNK, BS, HL, D), jnp.float32),
            pltpu.VMEM((B, 1, HL, D), jnp.float32),
            pltpu.VMEM((B, HL), jnp.float32),
            pltpu.VMEM((B, HL), jnp.float32),
            pltpu.VMEM((B, HL, D), jnp.float32),
            pltpu.VMEM((B, HL), jnp.float32),
            pltpu.VMEM((B, HL), jnp.float32),
            pltpu.VMEM((B, HL, D), jnp.float32),
            pltpu.VMEM((4, B, HL, D), jnp.float32),
            pltpu.SemaphoreType.DMA((2,)),
            pltpu.SemaphoreType.DMA((2,)),
            pltpu.SemaphoreType.DMA,
            pltpu.SemaphoreType.DMA((3,)),
            pltpu.SemaphoreType.DMA((3,)),
            pltpu.SemaphoreType.DMA((3,)),
            pltpu.SemaphoreType.DMA((4,)),
        ],
        compiler_params=pltpu.CompilerParams(
            collective_id=0,
        ),
    )(Q, K, V, w)


# device time: 28208 ns/iter; 3.2577x vs baseline; 1.0165x over previous
import jax
import jax.numpy as jnp
from jax import lax
from jax.experimental import pallas as pl
from jax.experimental.pallas import tpu as pltpu

B, H, D, BS = 8, 8, 128, 16
NB_LOCAL = 512
NB_Q = NB_LOCAL // 4
QCHUNK = 64
NQC = NB_Q // QCHUNK
QTOK = QCHUNK * BS
SCALE = D ** -0.5
NEG = -1e30


def kernel(Q, K, V, bt, lens):
    my_x = lax.axis_index("x")
    my_y = lax.axis_index("y")
    my_z = lax.axis_index("z")
    quarter = my_y * 2 + my_z

    page_ids = (my_x * NB_LOCAL + quarter * NB_Q
                + jnp.arange(NB_Q, dtype=jnp.int32))
    valid = jnp.arange(bt.shape[1], dtype=jnp.int32)[None, :] < lens[:, None]
    cnt = jnp.sum(
        (bt[:, :, None] == page_ids[None, None, :]) & valid[:, :, None],
        axis=1,
        dtype=jnp.int32,
    )
    w = jnp.repeat(cnt.astype(jnp.float32), BS, axis=1)

    def body(q_ref, k_hbm, v_hbm, w_ref, out_ref,
             kbuf, vbuf, m_acc, l_acc, o_acc, rm, rl, ro,
             ksems, vsems, rsend_sems, rrecv_sems):
        x = lax.axis_index("x")
        y = lax.axis_index("y")
        z = lax.axis_index("z")
        p0 = (y * 2 + z) * NB_Q

        descs = {}

        def start_copy(ci):
            slot = ci % 2
            dk = pltpu.make_async_copy(
                k_hbm.at[pl.ds(p0 + ci * QCHUNK, QCHUNK)],
                kbuf.at[slot], ksems.at[slot])
            dv = pltpu.make_async_copy(
                v_hbm.at[pl.ds(p0 + ci * QCHUNK, QCHUNK)],
                vbuf.at[slot], vsems.at[slot])
            dk.start()
            dv.start()
            descs[ci] = (dk, dv)

        start_copy(0)

        m_acc[...] = jnp.full((B, H), NEG, jnp.float32)
        l_acc[...] = jnp.zeros((B, H), jnp.float32)
        o_acc[...] = jnp.zeros((B, H, D), jnp.float32)
        qs = q_ref[...] * SCALE

        for ci in range(NQC):
            if ci + 1 < NQC:
                start_copy(ci + 1)
            dk, dv = descs[ci]
            dk.wait()
            dv.wait()
            slot = ci % 2

            wv = w_ref[:, ci * QTOK:(ci + 1) * QTOK]
            for h in range(H):
                qh = qs[:, 0, h, :]
                kh = kbuf[slot, :, :, h, :].reshape(QTOK, D)
                vh = vbuf[slot, :, :, h, :].reshape(QTOK, D)

                s = lax.dot_general(
                    qh, kh, (((1,), (1,)), ((), ())),
                    preferred_element_type=jnp.float32,
                )

                m_old = m_acc[:, h:h + 1]
                m_new = jnp.maximum(m_old, jnp.max(s, axis=1, keepdims=True))
                alpha = jnp.exp(m_old - m_new)
                p = wv * jnp.exp(s - m_new)
                l_new = (alpha * l_acc[:, h:h + 1]
                         + jnp.sum(p, axis=1, keepdims=True))
                pv = lax.dot_general(
                    p, vh, (((1,), (0,)), ((), ())),
                    preferred_element_type=jnp.float32,
                )
                o_acc[:, h, :] = alpha * o_acc[:, h, :] + pv
                m_acc[:, h:h + 1] = m_new
                l_acc[:, h:h + 1] = l_new

        my_q3 = x * 4 + y * 2 + z
        targets = [(xx, yy, zz)
                   for xx in range(2) for yy in range(2) for zz in range(2)]

        bsem = pltpu.get_barrier_semaphore()
        for dev in targets:
            pl.semaphore_signal(
                bsem, inc=1,
                device_id=dev, device_id_type=pl.DeviceIdType.MESH,
            )
        pl.semaphore_wait(bsem, len(targets))

        rdmas = []
        for t, dev in enumerate(targets):
            for i, (src, dst) in enumerate(
                [(m_acc, rm.at[my_q3]), (l_acc, rl.at[my_q3]),
                 (o_acc, ro.at[my_q3])]
            ):
                rdmas.append(pltpu.make_async_remote_copy(
                    src_ref=src, dst_ref=dst,
                    send_sem=rsend_sems.at[3 * t + i],
                    recv_sem=rrecv_sems.at[3 * my_q3 + i],
                    device_id=dev, device_id_type=pl.DeviceIdType.MESH,
                ))
        for rr in rdmas:
            rr.start()
        for rr in rdmas:
            rr.wait_send()
        for q in range(8):
            for i, dst in enumerate([rm.at[q], rl.at[q], ro.at[q]]):
                pltpu.make_async_remote_copy(
                    src_ref=m_acc if i == 0 else (l_acc if i == 1 else o_acc),
                    dst_ref=dst,
                    send_sem=rsend_sems.at[0],
                    recv_sem=rrecv_sems.at[3 * q + i],
                    device_id=(x, y, z),
                    device_id_type=pl.DeviceIdType.MESH,
                ).wait_recv()

        mm = rm[...]
        mx = jnp.max(mm, axis=0)
        aa = jnp.exp(mm - mx[None])
        lsum = jnp.sum(aa * rl[...], axis=0)
        o_tot = jnp.zeros((B, H, D), jnp.float32)
        for q in range(8):
            o_tot = o_tot + aa[q][:, :, None] * ro[q]

        out_ref[:, 0, :, :] = o_tot / lsum[:, :, None]

    return pl.pallas_call(
        body,
        out_shape=jax.ShapeDtypeStruct((B, 1, H, D), jnp.float32),
        in_specs=[
            pl.BlockSpec(memory_space=pltpu.VMEM),
            pl.BlockSpec(memory_space=pl.ANY),
            pl.BlockSpec(memory_space=pl.ANY),
            pl.BlockSpec(memory_space=pltpu.VMEM),
        ],
        out_specs=pl.BlockSpec(memory_space=pltpu.VMEM),
        scratch_shapes=[
            pltpu.VMEM((2, QCHUNK, BS, H, D), jnp.float32),
            pltpu.VMEM((2, QCHUNK, BS, H, D), jnp.float32),
            pltpu.VMEM((B, H), jnp.float32),
            pltpu.VMEM((B, H), jnp.float32),
            pltpu.VMEM((B, H, D), jnp.float32),
            pltpu.VMEM((8, B, H), jnp.float32),
            pltpu.VMEM((8, B, H), jnp.float32),
            pltpu.VMEM((8, B, H, D), jnp.float32),
            pltpu.SemaphoreType.DMA((2,)),
            pltpu.SemaphoreType.DMA((2,)),
            pltpu.SemaphoreType.DMA((24,)),
            pltpu.SemaphoreType.DMA((24,)),
        ],
        compiler_params=pltpu.CompilerParams(
            collective_id=0,
        ),
    )(Q, K, V, w)


# device time: 26713 ns/iter; 3.4400x vs baseline; 1.0560x over previous
import jax
import jax.numpy as jnp
from jax import lax
from jax.experimental import pallas as pl
from jax.experimental.pallas import tpu as pltpu

B, H, D, BS = 8, 8, 128, 16
NB_LOCAL = 512
NB_Q = NB_LOCAL // 4
QCHUNK = 64
NQC = NB_Q // QCHUNK
QTOK = QCHUNK * BS
SCALE = D ** -0.5
NEG = -1e30


def kernel(Q, K, V, bt, lens):
    lens2 = lens.reshape(B, 1)
    nbt = bt.shape[1]

    def body(q_ref, k_hbm, v_hbm, bt_ref, len_ref, out_ref,
             kbuf, vbuf, m_acc, l_acc, o_acc, rm, rl, ro,
             ksems, vsems, rsend_sems, rrecv_sems):
        x = lax.axis_index("x")
        y = lax.axis_index("y")
        z = lax.axis_index("z")
        p0 = (y * 2 + z) * NB_Q

        descs = {}

        def start_copy(ci):
            slot = ci % 2
            dk = pltpu.make_async_copy(
                k_hbm.at[pl.ds(p0 + ci * QCHUNK, QCHUNK)],
                kbuf.at[slot], ksems.at[slot])
            dv = pltpu.make_async_copy(
                v_hbm.at[pl.ds(p0 + ci * QCHUNK, QCHUNK)],
                vbuf.at[slot], vsems.at[slot])
            dk.start()
            dv.start()
            descs[ci] = (dk, dv)

        start_copy(0)

        m_acc[...] = jnp.full((B, H), NEG, jnp.float32)
        l_acc[...] = jnp.zeros((B, H), jnp.float32)
        o_acc[...] = jnp.zeros((B, H, D), jnp.float32)
        qs = q_ref[...] * SCALE

        pid0 = x * NB_LOCAL + (y * 2 + z) * NB_Q
        pids = pid0 + lax.broadcasted_iota(jnp.int32, (1, 1, NB_Q), 2)
        jidx = lax.broadcasted_iota(jnp.int32, (B, nbt), 1)
        btm = jnp.where(jidx < len_ref[...], bt_ref[...], -1)
        cnt = jnp.sum(
            (btm[:, :, None] == pids).astype(jnp.float32), axis=1
        )
        expand = (
            lax.broadcasted_iota(jnp.int32, (NB_Q, NB_Q * BS), 1) // BS
            == lax.broadcasted_iota(jnp.int32, (NB_Q, NB_Q * BS), 0)
        ).astype(jnp.float32)
        w_full = lax.dot_general(
            cnt, expand, (((1,), (0,)), ((), ())),
            preferred_element_type=jnp.float32,
        )

        for ci in range(NQC):
            if ci + 1 < NQC:
                start_copy(ci + 1)
            dk, dv = descs[ci]
            dk.wait()
            dv.wait()
            slot = ci % 2

            wv = w_full[:, ci * QTOK:(ci + 1) * QTOK]
            for h in range(H):
                qh = qs[:, 0, h, :]
                kh = kbuf[slot, :, :, h, :].reshape(QTOK, D)
                vh = vbuf[slot, :, :, h, :].reshape(QTOK, D)

                s = lax.dot_general(
                    qh, kh, (((1,), (1,)), ((), ())),
                    preferred_element_type=jnp.float32,
                )

                m_old = m_acc[:, h:h + 1]
                m_new = jnp.maximum(m_old, jnp.max(s, axis=1, keepdims=True))
                alpha = jnp.exp(m_old - m_new)
                p = wv * jnp.exp(s - m_new)
                l_new = (alpha * l_acc[:, h:h + 1]
                         + jnp.sum(p, axis=1, keepdims=True))
                pv = lax.dot_general(
                    p, vh, (((1,), (0,)), ((), ())),
                    preferred_element_type=jnp.float32,
                )
                o_acc[:, h, :] = alpha * o_acc[:, h, :] + pv
                m_acc[:, h:h + 1] = m_new
                l_acc[:, h:h + 1] = l_new

        my_q3 = x * 4 + y * 2 + z
        targets = [(xx, yy, zz)
                   for xx in range(2) for yy in range(2) for zz in range(2)]

        bsem = pltpu.get_barrier_semaphore()
        for dev in targets:
            pl.semaphore_signal(
                bsem, inc=1,
                device_id=dev, device_id_type=pl.DeviceIdType.MESH,
            )
        pl.semaphore_wait(bsem, len(targets))

        rdmas = []
        for t, dev in enumerate(targets):
            for i, (src, dst) in enumerate(
                [(m_acc, rm.at[my_q3]), (l_acc, rl.at[my_q3]),
                 (o_acc, ro.at[my_q3])]
            ):
                rdmas.append(pltpu.make_async_remote_copy(
                    src_ref=src, dst_ref=dst,
                    send_sem=rsend_sems.at[3 * t + i],
                    recv_sem=rrecv_sems.at[3 * my_q3 + i],
                    device_id=dev, device_id_type=pl.DeviceIdType.MESH,
                ))
        for rr in rdmas:
            rr.start()
        for rr in rdmas:
            rr.wait_send()
        for q in range(8):
            for i, dst in enumerate([rm.at[q], rl.at[q], ro.at[q]]):
                pltpu.make_async_remote_copy(
                    src_ref=m_acc if i == 0 else (l_acc if i == 1 else o_acc),
                    dst_ref=dst,
                    send_sem=rsend_sems.at[0],
                    recv_sem=rrecv_sems.at[3 * q + i],
                    device_id=(x, y, z),
                    device_id_type=pl.DeviceIdType.MESH,
                ).wait_recv()

        mm = rm[...]
        mx = jnp.max(mm, axis=0)
        aa = jnp.exp(mm - mx[None])
        lsum = jnp.sum(aa * rl[...], axis=0)
        o_tot = jnp.zeros((B, H, D), jnp.float32)
        for q in range(8):
            o_tot = o_tot + aa[q][:, :, None] * ro[q]

        out_ref[:, 0, :, :] = o_tot / lsum[:, :, None]

    return pl.pallas_call(
        body,
        out_shape=jax.ShapeDtypeStruct((B, 1, H, D), jnp.float32),
        in_specs=[
            pl.BlockSpec(memory_space=pltpu.VMEM),
            pl.BlockSpec(memory_space=pl.ANY),
            pl.BlockSpec(memory_space=pl.ANY),
            pl.BlockSpec(memory_space=pltpu.VMEM),
            pl.BlockSpec(memory_space=pltpu.VMEM),
        ],
        out_specs=pl.BlockSpec(memory_space=pltpu.VMEM),
        scratch_shapes=[
            pltpu.VMEM((2, QCHUNK, BS, H, D), jnp.float32),
            pltpu.VMEM((2, QCHUNK, BS, H, D), jnp.float32),
            pltpu.VMEM((B, H), jnp.float32),
            pltpu.VMEM((B, H), jnp.float32),
            pltpu.VMEM((B, H, D), jnp.float32),
            pltpu.VMEM((8, B, H), jnp.float32),
            pltpu.VMEM((8, B, H), jnp.float32),
            pltpu.VMEM((8, B, H, D), jnp.float32),
            pltpu.SemaphoreType.DMA((2,)),
            pltpu.SemaphoreType.DMA((2,)),
            pltpu.SemaphoreType.DMA((24,)),
            pltpu.SemaphoreType.DMA((24,)),
        ],
        compiler_params=pltpu.CompilerParams(
            collective_id=0,
        ),
    )(Q, K, V, bt, lens2)
